# baseline (speedup 1.0000x reference)
import jax
import jax.numpy as jnp
from jax.experimental import pallas as pl
from jax.experimental.pallas import tpu as pltpu
from jax.experimental.pallas import tpu_sc as plsc

_RB = 8
_CB = 256
_V = 16


def _tc_body(concat_ref, table_ref, out_ref):
    out_ref[...] = concat_ref[...] + table_ref[...][None, :, :]


def _tc_add(concat, table):
    batch, seq, hidden = concat.shape
    bs = 256
    return pl.pallas_call(
        _tc_body,
        grid=(seq // bs,),
        in_specs=[
            pl.BlockSpec((batch, bs, hidden), lambda i: (0, i, 0)),
            pl.BlockSpec((bs, hidden), lambda i: (i, 0)),
        ],
        out_specs=pl.BlockSpec((batch, bs, hidden), lambda i: (0, i, 0)),
        out_shape=jax.ShapeDtypeStruct((batch, seq, hidden), concat.dtype),
    )(concat, table)


def _sc_body(concat_hbm, table_hbm, out_hbm):
    rows, hidden = concat_hbm.shape

    def body(c_vmem, t_vmem, o_vmem):
        @pl.loop(0, _RB)
        def _(r):
            @pl.loop(0, _CB, step=_V)
            def _(c):
                slc = (r, pl.ds(c, _V))
                o_vmem.at[*slc][...] = c_vmem.at[*slc][...] + t_vmem.at[*slc][...]

    pltpu.emit_pipeline(
        body,
        grid=(rows // _RB, hidden // _CB),
        in_specs=[
            pl.BlockSpec((_RB, _CB), index_map=lambda i, j: (i, j)),
            pl.BlockSpec((_RB, _CB), index_map=lambda i, j: (i, j)),
        ],
        out_specs=[pl.BlockSpec((_RB, _CB), index_map=lambda i, j: (i, j))],
        core_axis_name=("core", "subcore"),
        dimension_semantics=(pltpu.PARALLEL, pltpu.PARALLEL),
    )(concat_hbm, table_hbm, out_hbm)


def _sc_add(concat2d, table):
    mesh = plsc.VectorSubcoreMesh(core_axis_name="core", subcore_axis_name="subcore")
    sc_fn = pl.kernel(
        _sc_body,
        out_type=jax.ShapeDtypeStruct(concat2d.shape, concat2d.dtype),
        mesh=mesh,
        scratch_types=[],
    )
    return sc_fn(concat2d, table)


def kernel(concat_embeddings, position_table):
    batch, seq, hidden = concat_embeddings.shape
    table = position_table[:seq]
    tc_out = _tc_add(concat_embeddings[: batch - 1], table)
    sc_out = _sc_add(concat_embeddings[batch - 1], table)
    return jnp.concatenate([tc_out, sc_out[None]], axis=0)
